# baseline (speedup 1.0000x reference)
import functools

import jax
import jax.numpy as jnp
from jax import lax
from jax.experimental import pallas as pl
from jax.experimental.pallas import tpu as pltpu
from jax.experimental.pallas import tpu_sc as plsc

_D = 64
_NW = 32
_XROWS_PER_CHUNK = 8
_NSLOT = 4


@functools.partial(jax.jit, static_argnums=(2, 3))
def _sc_embedding_gather(x32, weight, b, s):
    xrows_per_w = b // _NW
    n_chunks = xrows_per_w // _XROWS_PER_CHUNK
    mesh = plsc.VectorSubcoreMesh(core_axis_name="c", subcore_axis_name="s")

    @functools.partial(
        pl.kernel,
        out_type=jax.ShapeDtypeStruct((b, s, _D), jnp.float32),
        mesh=mesh,
        scratch_types=[
            pltpu.VMEM((xrows_per_w, s), jnp.int32),
            pltpu.VMEM((_NSLOT, _XROWS_PER_CHUNK, s, _D), jnp.float32),
            pltpu.SemaphoreType.DMA((_NSLOT,)),
            pltpu.SemaphoreType.DMA((_NSLOT,)),
        ],
        compiler_params=pltpu.CompilerParams(use_tc_tiling_on_sc=False),
    )
    def k(table_hbm, x_hbm, out_hbm, idx_v, rows_v, gsem, ssem):
        wid = lax.axis_index("s") * 2 + lax.axis_index("c")
        base_x = wid * xrows_per_w
        pltpu.sync_copy(x_hbm.at[pl.ds(base_x, xrows_per_w)], idx_v)

        def fire_gathers(c, slot):
            ds = []
            for j in range(_XROWS_PER_CHUNK):
                ds.append(
                    pltpu.async_copy(
                        table_hbm.at[idx_v.at[c * _XROWS_PER_CHUNK + j]],
                        rows_v.at[slot].at[j],
                        gsem.at[slot],
                    )
                )
            return ds

        def drain_gathers(c, slot):
            for j in range(_XROWS_PER_CHUNK):
                pltpu.make_async_copy(
                    table_hbm.at[idx_v.at[c * _XROWS_PER_CHUNK + j]],
                    rows_v.at[slot].at[j],
                    gsem.at[slot],
                ).wait()

        def fire_scatter(c, slot):
            return pltpu.async_copy(
                rows_v.at[slot],
                out_hbm.at[pl.ds(base_x + c * _XROWS_PER_CHUNK, _XROWS_PER_CHUNK)],
                ssem.at[slot],
            )

        def drain_scatter(c, slot):
            pltpu.make_async_copy(
                rows_v.at[slot],
                out_hbm.at[pl.ds(base_x + c * _XROWS_PER_CHUNK, _XROWS_PER_CHUNK)],
                ssem.at[slot],
            ).wait()

        for c in range(_NSLOT):
            fire_gathers(c, c)

        def body(c, _):
            slot = lax.rem(c, _NSLOT)
            drain_gathers(c, slot)
            fire_scatter(c, slot)
            drain_scatter(c, slot)
            fire_gathers(c + _NSLOT, slot)
            return 0

        lax.fori_loop(0, n_chunks - _NSLOT, body, 0)

        for c in range(n_chunks - _NSLOT, n_chunks):
            slot = c % _NSLOT
            drain_gathers(c, slot)
            fire_scatter(c, slot)
        for c in range(n_chunks - _NSLOT, n_chunks):
            drain_scatter(c, c % _NSLOT)

    return k(weight, x32)


def kernel(x, weight):
    b, s = x.shape
    out = _sc_embedding_gather(x.astype(jnp.int32), weight, b, s)
    return out
